# baseline (speedup 1.0000x reference)
import functools

import jax
import jax.numpy as jnp
from jax import lax
from jax.experimental import pallas as pl
from jax.experimental.pallas import tpu as pltpu
from jax.experimental.pallas import tpu_sc as plsc

NC = 2
NS = 16
NW = NC * NS
GB = 4
M = 4
K = 2
H = 4
TG = 8


def _sc_part(h, rh, sp, d):
    nbp = rh // NW
    nq = nbp // GB
    assert nq % M == 0 and nq >= 2 * M
    mesh = plsc.VectorSubcoreMesh(
        core_axis_name="c", subcore_axis_name="s",
        num_cores=NC, num_subcores=NS,
    )

    @functools.partial(
        pl.kernel,
        out_type=jax.ShapeDtypeStruct((rh * sp, d), jnp.float32),
        mesh=mesh,
        scratch_types=[
            pltpu.VMEM((nbp, sp), jnp.int32),
            [pltpu.VMEM((GB * sp, d), jnp.float32) for _ in range(M)],
            [pltpu.SemaphoreType.DMA for _ in range(M)],
            [pltpu.SemaphoreType.DMA for _ in range(M)],
        ],
    )
    def emb_kernel(table_hbm, idx_hbm, out_hbm, idx_v, rows, gsem, wsem):
        wid = lax.axis_index("s") * NC + lax.axis_index("c")
        pltpu.sync_copy(idx_hbm.at[pl.ds(h * rh + wid * nbp, nbp)], idx_v)

        def gathers(q, b):
            for u in range(GB):
                pltpu.async_copy(
                    table_hbm.at[idx_v.at[q * GB + u]],
                    rows[b].at[pl.ds(u * sp, sp)], gsem[b])

        def wait_gathers(q, b):
            for u in range(GB):
                pltpu.make_async_copy(
                    table_hbm.at[idx_v.at[q * GB + u]],
                    rows[b].at[pl.ds(u * sp, sp)], gsem[b]).wait()

        def write(q, b):
            pltpu.async_copy(
                rows[b],
                out_hbm.at[pl.ds((wid * nbp + q * GB) * sp, GB * sp)],
                wsem[b])

        def wait_write(q, b):
            pltpu.make_async_copy(
                rows[b],
                out_hbm.at[pl.ds((wid * nbp + q * GB) * sp, GB * sp)],
                wsem[b]).wait()

        for b in range(M):
            gathers(b, b)
            if b >= K:
                qq = b - K
                wait_gathers(qq, qq)
                write(qq, qq)

        def round_body(r, _):
            for b in range(M):
                q = r * M + b
                wait_write(q - M, b)
                gathers(q, b)
                bb = (b - K) % M
                wait_gathers(q - K, bb)
                write(q - K, bb)
            return ()

        lax.fori_loop(1, nq // M, round_body, ())

        for qq in range(nq - K, nq):
            bb = qq % M
            wait_gathers(qq, bb)
            write(qq, bb)
        for b in range(M):
            wait_write(nq - M + b, b)

    return emb_kernel


def _tc_relayout(h, rh, n, s, sp, d):
    blocks = rh // TG

    def body(*refs):
        in_ref, out_ref = refs[0], refs[-1]
        out_ref[...] = in_ref[...].reshape(TG, sp, d)[:, :s, :]

    in_specs = [pl.BlockSpec((TG * sp, d), lambda i: (i, 0))]
    if h > 0:
        in_specs.append(pl.BlockSpec(memory_space=pltpu.MemorySpace.HBM))
    return pl.pallas_call(
        body,
        grid=(blocks,),
        in_specs=in_specs,
        out_specs=pl.BlockSpec(
            (TG, s, d), lambda i, _h=h, _b=blocks: (_h * _b + i, 0, 0)),
        out_shape=jax.ShapeDtypeStruct((n, s, d), jnp.float32),
        input_output_aliases=({1: 0} if h > 0 else {}),
        name=f"relayout_part{h}",
    )


@functools.partial(jax.jit, static_argnames=("n", "s", "d"))
def _emb_lookup(xi, table, *, n, s, d):
    rh = n // H
    sp = xi.shape[1]
    parts = [_sc_part(h, rh, sp, d)(table, xi) for h in range(H)]
    out = _tc_relayout(0, rh, n, s, sp, d)(parts[0])
    for h in range(1, H):
        out = _tc_relayout(h, rh, n, s, sp, d)(parts[h], out)
    return out


def kernel(x, table):
    n, s = x.shape
    d = table.shape[1]
    sp = (s + 7) // 8 * 8
    assert n % (NW * GB * H) == 0 and (n // H) % (NW * TG) == 0
    xi = x.astype(jnp.int32)
    if sp != s:
        xi = jnp.pad(xi, ((0, 0), (0, sp - s)))
    return _emb_lookup(xi, table, n=n, s=s, d=d)


# speedup vs baseline: 4.1395x; 4.1395x over previous
import functools

import jax
import jax.numpy as jnp
from jax import lax
from jax.experimental import pallas as pl
from jax.experimental.pallas import tpu as pltpu
from jax.experimental.pallas import tpu_sc as plsc

NC = 2
NS = 16
NW = NC * NS
GB = 4
M = 4
K = 2
H = 1
TG = 16


def _sc_part(h, rh, sp, d):
    nbp = rh // NW
    nq = nbp // GB
    assert nq % M == 0 and nq >= 2 * M
    mesh = plsc.VectorSubcoreMesh(
        core_axis_name="c", subcore_axis_name="s",
        num_cores=NC, num_subcores=NS,
    )

    @functools.partial(
        pl.kernel,
        out_type=jax.ShapeDtypeStruct((rh * sp, d), jnp.float32),
        mesh=mesh,
        scratch_types=[
            pltpu.VMEM((nbp, sp), jnp.int32),
            [pltpu.VMEM((GB * sp, d), jnp.float32) for _ in range(M)],
            [pltpu.SemaphoreType.DMA for _ in range(M)],
            [pltpu.SemaphoreType.DMA for _ in range(M)],
        ],
    )
    def emb_kernel(table_hbm, idx_hbm, out_hbm, idx_v, rows, gsem, wsem):
        wid = lax.axis_index("s") * NC + lax.axis_index("c")
        pltpu.sync_copy(idx_hbm.at[pl.ds(h * rh + wid * nbp, nbp)], idx_v)

        def gathers(q, b):
            for u in range(GB):
                pltpu.async_copy(
                    table_hbm.at[idx_v.at[q * GB + u]],
                    rows[b].at[pl.ds(u * sp, sp)], gsem[b])

        def wait_gathers(q, b):
            for u in range(GB):
                pltpu.make_async_copy(
                    table_hbm.at[idx_v.at[q * GB + u]],
                    rows[b].at[pl.ds(u * sp, sp)], gsem[b]).wait()

        def write(q, b):
            pltpu.async_copy(
                rows[b],
                out_hbm.at[pl.ds((wid * nbp + q * GB) * sp, GB * sp)],
                wsem[b])

        def wait_write(q, b):
            pltpu.make_async_copy(
                rows[b],
                out_hbm.at[pl.ds((wid * nbp + q * GB) * sp, GB * sp)],
                wsem[b]).wait()

        for b in range(M):
            gathers(b, b)
            if b >= K:
                qq = b - K
                wait_gathers(qq, qq)
                write(qq, qq)

        def round_body(r, _):
            for b in range(M):
                q = r * M + b
                wait_write(q - M, b)
                gathers(q, b)
                bb = (b - K) % M
                wait_gathers(q - K, bb)
                write(q - K, bb)
            return ()

        lax.fori_loop(1, nq // M, round_body, ())

        for qq in range(nq - K, nq):
            bb = qq % M
            wait_gathers(qq, bb)
            write(qq, bb)
        for b in range(M):
            wait_write(nq - M + b, b)

    return emb_kernel


def _tc_relayout(h, rh, n, s, sp, d):
    blocks = rh // TG

    def body(*refs):
        in_ref, out_ref = refs[0], refs[-1]
        out_ref[...] = in_ref[...].reshape(TG, sp, d)[:, :s, :]

    in_specs = [pl.BlockSpec((TG * sp, d), lambda i: (i, 0))]
    if h > 0:
        in_specs.append(pl.BlockSpec(memory_space=pltpu.MemorySpace.HBM))
    return pl.pallas_call(
        body,
        grid=(blocks,),
        in_specs=in_specs,
        out_specs=pl.BlockSpec(
            (TG, s, d), lambda i, _h=h, _b=blocks: (_h * _b + i, 0, 0)),
        out_shape=jax.ShapeDtypeStruct((n, s, d), jnp.float32),
        input_output_aliases=({1: 0} if h > 0 else {}),
        name=f"relayout_part{h}",
    )


@functools.partial(jax.jit, static_argnames=("n", "s", "d"))
def _emb_lookup(xi, table, *, n, s, d):
    rh = n // H
    sp = xi.shape[1]
    parts = [_sc_part(h, rh, sp, d)(table, xi) for h in range(H)]
    out = _tc_relayout(0, rh, n, s, sp, d)(parts[0])
    for h in range(1, H):
        out = _tc_relayout(h, rh, n, s, sp, d)(parts[h], out)
    return out


def kernel(x, table):
    n, s = x.shape
    d = table.shape[1]
    sp = (s + 7) // 8 * 8
    assert n % (NW * GB * H) == 0 and (n // H) % (NW * TG) == 0
    xi = x.astype(jnp.int32)
    if sp != s:
        xi = jnp.concatenate([xi, xi[:, : sp - s]], axis=1)
    return _emb_lookup(xi, table, n=n, s=s, d=d)


# speedup vs baseline: 8.8374x; 2.1349x over previous
import functools

import jax
import jax.numpy as jnp
from jax import lax
from jax.experimental import pallas as pl
from jax.experimental.pallas import tpu as pltpu
from jax.experimental.pallas import tpu_sc as plsc

NC = 2
NS = 16
NW = NC * NS
GB = 4
M = 4
K = 2


@functools.partial(jax.jit, static_argnames=("nb", "s", "d"))
def _emb_lookup(xi, table, *, nb, s, d):
    nq = nb // GB
    mesh = plsc.VectorSubcoreMesh(
        core_axis_name="c", subcore_axis_name="s",
        num_cores=NC, num_subcores=NS,
    )

    @functools.partial(
        pl.kernel,
        out_type=jax.ShapeDtypeStruct((NW * nb, s, d), jnp.float32),
        mesh=mesh,
        scratch_types=[
            pltpu.VMEM((nb, s), jnp.int32),
            [pltpu.VMEM((GB, s, d), jnp.float32) for _ in range(M)],
            [pltpu.SemaphoreType.DMA for _ in range(M)],
            [pltpu.SemaphoreType.DMA for _ in range(M)],
        ],
    )
    def emb_kernel(table_hbm, idx_hbm, out_hbm, idx_v, rows, gsem, wsem):
        wid = lax.axis_index("s") * NC + lax.axis_index("c")
        base = wid * nb
        pltpu.sync_copy(idx_hbm.at[pl.ds(base, nb)], idx_v)

        def gathers(q, b):
            for u in range(GB):
                pltpu.async_copy(
                    table_hbm.at[idx_v.at[q * GB + u]], rows[b].at[u],
                    gsem[b])

        def wait_gathers(q, b):
            for u in range(GB):
                pltpu.make_async_copy(
                    table_hbm.at[idx_v.at[q * GB + u]], rows[b].at[u],
                    gsem[b]).wait()

        def write(q, b):
            pltpu.async_copy(
                rows[b], out_hbm.at[pl.ds(base + q * GB, GB)], wsem[b])

        def wait_write(q, b):
            pltpu.make_async_copy(
                rows[b], out_hbm.at[pl.ds(base + q * GB, GB)],
                wsem[b]).wait()

        for b in range(M):
            gathers(b, b)
            if b >= K:
                qq = b - K
                wait_gathers(qq, qq)
                write(qq, qq)

        def round_body(r, _):
            for b in range(M):
                q = r * M + b
                wait_write(q - M, b)
                gathers(q, b)
                bb = (b - K) % M
                wait_gathers(q - K, bb)
                write(q - K, bb)
            return ()

        lax.fori_loop(1, nq // M, round_body, ())

        for qq in range(nq - K, nq):
            bb = qq % M
            wait_gathers(qq, bb)
            write(qq, bb)
        for b in range(M):
            wait_write(nq - M + b, b)

    return emb_kernel(table, xi)


def kernel(x, table):
    n, s = x.shape
    d = table.shape[1]
    assert n % (NW * GB) == 0
    nb = n // NW
    nq = nb // GB
    assert nq % M == 0 and nq >= 2 * M
    xi = x.astype(jnp.int32)
    return _emb_lookup(xi, table, nb=nb, s=s, d=d)
